# speedup vs baseline: 1.3739x; 1.3739x over previous
import functools

import jax
import jax.numpy as jnp
from jax.experimental import pallas as pl
from jax.experimental.pallas import tpu as pltpu

EPS = 1e-6
PB = 4096


def _matcher_kernel(num_p_blocks, logits_ref, labels_ref, x_ref, y_ref,
                    out_ref, acc_xy, acc_sxy, acc_sp, acc_sg, acc_ys):
    p = pl.program_id(1)

    @pl.when(p == 0)
    def _init():
        acc_xy[...] = jnp.zeros_like(acc_xy)
        acc_sxy[...] = jnp.zeros_like(acc_sxy)
        acc_sp[...] = jnp.zeros_like(acc_sp)
        acc_sg[...] = jnp.zeros_like(acc_sg)
        acc_ys[...] = jnp.zeros_like(acc_ys)

    xb = x_ref[0]
    yb = y_ref[0]

    e = jnp.exp(-jnp.abs(xb))
    one_p_e = 1.0 + e
    sp = jnp.maximum(xb, 0.0) + jnp.log(one_p_e)
    sg = jnp.where(xb >= 0.0, 1.0, e) / one_p_e

    x16 = xb.astype(jnp.bfloat16)
    sg16 = sg.astype(jnp.bfloat16)
    y16 = yb.astype(jnp.bfloat16)

    contract = (((1,), (1,)), ((), ()))
    acc_xy[...] += jax.lax.dot_general(
        x16, y16, contract, preferred_element_type=jnp.float32)
    acc_sxy[...] += jax.lax.dot_general(
        sg16, y16, contract, preferred_element_type=jnp.float32)
    acc_sp[...] += jnp.sum(sp, axis=1, keepdims=True)
    acc_sg[...] += jnp.sum(sg, axis=1, keepdims=True)
    ones8 = jnp.ones((8, PB), jnp.bfloat16)
    ys8 = jax.lax.dot_general(
        ones8, y16, contract, preferred_element_type=jnp.float32)
    acc_ys[...] += ys8[0:1, :]

    @pl.when(p == num_p_blocks - 1)
    def _finalize():
        n_t = acc_xy.shape[1]
        inv_p = 1.0 / jnp.float32(num_p_blocks * PB)
        logits = logits_ref[0]
        m = jnp.max(logits, axis=-1, keepdims=True)
        ex = jnp.exp(logits - m)
        prob = ex / jnp.sum(ex, axis=-1, keepdims=True)
        lab = labels_ref[0]
        num_classes = logits.shape[-1]
        iota_c = jax.lax.broadcasted_iota(jnp.int32, (num_classes, n_t), 0)
        onehot = (iota_c == lab).astype(jnp.float32)
        cost_class = jax.lax.dot_general(
            prob, onehot, (((1,), (0,)), ((), ())),
            preferred_element_type=jnp.float32)

        bce = (acc_sp[...] - acc_xy[...]) * inv_p
        denom = acc_sg[...] + acc_ys[...] + EPS
        dice = 1.0 - 2.0 * acc_sxy[...] / denom
        out_ref[0] = bce + dice - cost_class


def kernel(pred_logits, pred_masks, tgt_labels, tgt_masks):
    bs, Q, C = pred_logits.shape
    T = tgt_masks.shape[1]
    P = pred_masks.shape[-2] * pred_masks.shape[-1]
    num_p_blocks = P // PB

    x = pred_masks.reshape(bs, Q, P)
    y = tgt_masks.reshape(bs, T, P)
    labels = tgt_labels.astype(jnp.int32).reshape(bs, 1, T)

    grid = (bs, num_p_blocks)
    out = pl.pallas_call(
        functools.partial(_matcher_kernel, num_p_blocks),
        grid=grid,
        in_specs=[
            pl.BlockSpec((1, Q, C), lambda b, p: (b, 0, 0)),
            pl.BlockSpec((1, 1, T), lambda b, p: (b, 0, 0)),
            pl.BlockSpec((1, Q, PB), lambda b, p: (b, 0, p)),
            pl.BlockSpec((1, T, PB), lambda b, p: (b, 0, p)),
        ],
        out_specs=pl.BlockSpec((1, Q, T), lambda b, p: (b, 0, 0)),
        out_shape=jax.ShapeDtypeStruct((bs, Q, T), jnp.float32),
        scratch_shapes=[
            pltpu.VMEM((Q, T), jnp.float32),
            pltpu.VMEM((Q, T), jnp.float32),
            pltpu.VMEM((Q, 1), jnp.float32),
            pltpu.VMEM((Q, 1), jnp.float32),
            pltpu.VMEM((1, T), jnp.float32),
        ],
        compiler_params=pltpu.CompilerParams(
            dimension_semantics=("parallel", "arbitrary")),
    )(pred_logits, labels, x, y)
    return out


# speedup vs baseline: 1.6692x; 1.2150x over previous
import functools

import jax
import jax.numpy as jnp
from jax.experimental import pallas as pl
from jax.experimental.pallas import tpu as pltpu

EPS = 1e-6
BH = 16


def _matcher_kernel(num_h_blocks, Q, T, logits_ref, labels_ref, x_ref, y_ref,
                    out_ref, acc):
    h = pl.program_id(1)

    @pl.when(h == 0)
    def _init():
        acc[...] = jnp.zeros_like(acc)

    xb = x_ref[...]
    yb = y_ref[...]
    W = xb.shape[-1]

    e = jnp.exp(-jnp.abs(xb))
    one_p_e = 1.0 + e
    sp = jnp.maximum(xb, 0.0) + jnp.log(one_p_e)
    sg = jnp.where(xb >= 0.0, 1.0, e) / one_p_e

    x16 = xb.astype(jnp.bfloat16)
    sg16 = sg.astype(jnp.bfloat16)
    sp16 = sp.astype(jnp.bfloat16)
    y16 = yb.astype(jnp.bfloat16)

    lhs = jnp.concatenate(
        [x16, sg16, sp16, jnp.ones((4, BH, W), jnp.bfloat16)], axis=0)
    rhs = jnp.concatenate(
        [y16, jnp.ones((1, BH, W), jnp.bfloat16)], axis=0)
    contract = (((2,), (2,)), ((1,), (1,)))
    bdot = jax.lax.dot_general(
        lhs, rhs, contract, preferred_element_type=jnp.float32)
    acc[...] += jnp.sum(bdot, axis=0)

    @pl.when(h == num_h_blocks - 1)
    def _finalize():
        inv_p = 1.0 / jnp.float32(num_h_blocks * BH * W)
        logits = logits_ref[0]
        m = jnp.max(logits, axis=-1, keepdims=True)
        ex = jnp.exp(logits - m)
        prob = ex / jnp.sum(ex, axis=-1, keepdims=True)
        lab = labels_ref[0]
        num_classes = logits.shape[-1]
        iota_c = jax.lax.broadcasted_iota(jnp.int32, (num_classes, T), 0)
        onehot = (iota_c == lab).astype(jnp.float32)
        cost_class = jax.lax.dot_general(
            prob, onehot, (((1,), (0,)), ((), ())),
            preferred_element_type=jnp.float32)

        xy = acc[0:Q, 0:T]
        sgy = acc[Q:2 * Q, 0:T]
        sg_sum = acc[Q:2 * Q, T:T + 1]
        sp_sum = acc[2 * Q:3 * Q, T:T + 1]
        y_sum = acc[3 * Q:3 * Q + 1, 0:T]

        bce = (sp_sum - xy) * inv_p
        denom = sg_sum + y_sum + EPS
        dice = 1.0 - 2.0 * sgy / denom
        out_ref[0] = bce + dice - cost_class


def kernel(pred_logits, pred_masks, tgt_labels, tgt_masks):
    bs, Q, C = pred_logits.shape
    T = tgt_masks.shape[1]
    H, W = pred_masks.shape[-2:]
    num_h_blocks = H // BH

    x3 = pred_masks.reshape(bs * Q, H, W)
    y3 = tgt_masks.reshape(bs * T, H, W)
    labels = tgt_labels.astype(jnp.int32).reshape(bs, 1, T)

    grid = (bs, num_h_blocks)
    out = pl.pallas_call(
        functools.partial(_matcher_kernel, num_h_blocks, Q, T),
        grid=grid,
        in_specs=[
            pl.BlockSpec((1, Q, C), lambda b, h: (b, 0, 0)),
            pl.BlockSpec((1, 1, T), lambda b, h: (b, 0, 0)),
            pl.BlockSpec((Q, BH, W), lambda b, h: (b, h, 0)),
            pl.BlockSpec((T, BH, W), lambda b, h: (b, h, 0)),
        ],
        out_specs=pl.BlockSpec((1, Q, T), lambda b, h: (b, 0, 0)),
        out_shape=jax.ShapeDtypeStruct((bs, Q, T), jnp.float32),
        scratch_shapes=[
            pltpu.VMEM((3 * Q + 4, T + 1), jnp.float32),
        ],
        compiler_params=pltpu.CompilerParams(
            dimension_semantics=("parallel", "arbitrary")),
    )(pred_logits, labels, x3, y3)
    return out


# speedup vs baseline: 2.5967x; 1.5556x over previous
import functools

import jax
import jax.numpy as jnp
from jax.experimental import pallas as pl
from jax.experimental.pallas import tpu as pltpu

EPS = 1e-6
BH = 16


def _matcher_kernel(num_h_blocks, Q, T, logits_ref, labels_ref, x_ref, y_ref,
                    out_ref, acc):
    h = pl.program_id(1)

    @pl.when(h == 0)
    def _init():
        acc[...] = jnp.zeros_like(acc)

    xb = x_ref[...]
    yb = y_ref[...]
    W = xb.shape[-1]

    e = jnp.exp(-jnp.abs(xb))
    one_p_e = 1.0 + e
    sp = jnp.maximum(xb, 0.0) + jnp.log(one_p_e)
    sg = jnp.where(xb >= 0.0, 1.0, e) / one_p_e

    x16 = xb.astype(jnp.bfloat16)
    sg16 = sg.astype(jnp.bfloat16)
    sp16 = sp.astype(jnp.bfloat16)
    y16 = yb.astype(jnp.bfloat16)

    lhs = jnp.concatenate(
        [x16, sg16, sp16, jnp.ones((4, BH, W), jnp.bfloat16)], axis=0)
    rhs = jnp.concatenate(
        [y16, jnp.ones((1, BH, W), jnp.bfloat16)], axis=0)
    lhs_t = jax.lax.transpose(lhs, (1, 0, 2))
    rhs_t = jax.lax.transpose(rhs, (1, 0, 2))
    contract = (((1,), (1,)), ((), ()))
    total = jax.lax.dot_general(
        lhs_t[0], rhs_t[0], contract, preferred_element_type=jnp.float32)
    for hh in range(1, BH):
        total += jax.lax.dot_general(
            lhs_t[hh], rhs_t[hh], contract,
            preferred_element_type=jnp.float32)
    acc[...] += total

    @pl.when(h == num_h_blocks - 1)
    def _finalize():
        inv_p = 1.0 / jnp.float32(num_h_blocks * BH * W)
        logits = logits_ref[0]
        m = jnp.max(logits, axis=-1, keepdims=True)
        ex = jnp.exp(logits - m)
        prob = ex / jnp.sum(ex, axis=-1, keepdims=True)
        lab = labels_ref[0]
        num_classes = logits.shape[-1]
        iota_c = jax.lax.broadcasted_iota(jnp.int32, (num_classes, T), 0)
        onehot = (iota_c == lab).astype(jnp.float32)
        cost_class = jax.lax.dot_general(
            prob, onehot, (((1,), (0,)), ((), ())),
            preferred_element_type=jnp.float32)

        xy = acc[0:Q, 0:T]
        sgy = acc[Q:2 * Q, 0:T]
        sg_sum = acc[Q:2 * Q, T:T + 1]
        sp_sum = acc[2 * Q:3 * Q, T:T + 1]
        y_sum = acc[3 * Q:3 * Q + 1, 0:T]

        bce = (sp_sum - xy) * inv_p
        denom = sg_sum + y_sum + EPS
        dice = 1.0 - 2.0 * sgy / denom
        out_ref[0] = bce + dice - cost_class


def kernel(pred_logits, pred_masks, tgt_labels, tgt_masks):
    bs, Q, C = pred_logits.shape
    T = tgt_masks.shape[1]
    H, W = pred_masks.shape[-2:]
    num_h_blocks = H // BH

    x3 = pred_masks.reshape(bs * Q, H, W)
    y3 = tgt_masks.reshape(bs * T, H, W)
    labels = tgt_labels.astype(jnp.int32).reshape(bs, 1, T)

    grid = (bs, num_h_blocks)
    out = pl.pallas_call(
        functools.partial(_matcher_kernel, num_h_blocks, Q, T),
        grid=grid,
        in_specs=[
            pl.BlockSpec((1, Q, C), lambda b, h: (b, 0, 0)),
            pl.BlockSpec((1, 1, T), lambda b, h: (b, 0, 0)),
            pl.BlockSpec((Q, BH, W), lambda b, h: (b, h, 0)),
            pl.BlockSpec((T, BH, W), lambda b, h: (b, h, 0)),
        ],
        out_specs=pl.BlockSpec((1, Q, T), lambda b, h: (b, 0, 0)),
        out_shape=jax.ShapeDtypeStruct((bs, Q, T), jnp.float32),
        scratch_shapes=[
            pltpu.VMEM((3 * Q + 4, T + 1), jnp.float32),
        ],
        compiler_params=pltpu.CompilerParams(
            dimension_semantics=("parallel", "arbitrary")),
    )(pred_logits, labels, x3, y3)
    return out


# speedup vs baseline: 2.6642x; 1.0260x over previous
import functools

import jax
import jax.numpy as jnp
from jax.experimental import pallas as pl
from jax.experimental.pallas import tpu as pltpu

EPS = 1e-6
BH = 16


def _matcher_kernel(num_h_blocks, Q, T, logits_ref, labels_ref, x_ref, y_ref,
                    out_ref, acc):
    h = pl.program_id(1)

    @pl.when(h == 0)
    def _init():
        acc[...] = jnp.zeros_like(acc)

    xb = x_ref[...]
    yb = y_ref[...]
    W = xb.shape[-1]

    e = jnp.exp2(jnp.abs(xb) * (-1.4426950408889634))
    one_p_e = 1.0 + e
    sp = jnp.maximum(xb, 0.0) + jnp.log(one_p_e)
    sg = jnp.where(xb >= 0.0, 1.0, e) / one_p_e

    x16 = xb.astype(jnp.bfloat16)
    sg16 = sg.astype(jnp.bfloat16)
    sp16 = sp.astype(jnp.bfloat16)
    y16 = yb.astype(jnp.bfloat16)

    lhs = jnp.concatenate(
        [x16, sg16, sp16, jnp.ones((4, BH, W), jnp.bfloat16)], axis=0)
    rhs = jnp.concatenate(
        [y16, jnp.ones((1, BH, W), jnp.bfloat16)], axis=0)
    lhs_t = jax.lax.transpose(lhs, (1, 0, 2))
    rhs_t = jax.lax.transpose(rhs, (1, 0, 2))
    lhs_f = jnp.concatenate([lhs_t[hh] for hh in range(BH)], axis=1)
    rhs_f = jnp.concatenate([rhs_t[hh] for hh in range(BH)], axis=1)
    contract = (((1,), (1,)), ((), ()))
    acc[...] += jax.lax.dot_general(
        lhs_f, rhs_f, contract, preferred_element_type=jnp.float32)

    @pl.when(h == num_h_blocks - 1)
    def _finalize():
        inv_p = 1.0 / jnp.float32(num_h_blocks * BH * W)
        logits = logits_ref[0]
        m = jnp.max(logits, axis=-1, keepdims=True)
        ex = jnp.exp(logits - m)
        prob = ex / jnp.sum(ex, axis=-1, keepdims=True)
        lab = labels_ref[0]
        num_classes = logits.shape[-1]
        iota_c = jax.lax.broadcasted_iota(jnp.int32, (num_classes, T), 0)
        onehot = (iota_c == lab).astype(jnp.float32)
        cost_class = jax.lax.dot_general(
            prob, onehot, (((1,), (0,)), ((), ())),
            preferred_element_type=jnp.float32)

        xy = acc[0:Q, 0:T]
        sgy = acc[Q:2 * Q, 0:T]
        sg_sum = acc[Q:2 * Q, T:T + 1]
        sp_sum = acc[2 * Q:3 * Q, T:T + 1]
        y_sum = acc[3 * Q:3 * Q + 1, 0:T]

        bce = (sp_sum - xy) * inv_p
        denom = sg_sum + y_sum + EPS
        dice = 1.0 - 2.0 * sgy / denom
        out_ref[0] = bce + dice - cost_class


def kernel(pred_logits, pred_masks, tgt_labels, tgt_masks):
    bs, Q, C = pred_logits.shape
    T = tgt_masks.shape[1]
    H, W = pred_masks.shape[-2:]
    num_h_blocks = H // BH

    x3 = pred_masks.reshape(bs * Q, H, W)
    y3 = tgt_masks.reshape(bs * T, H, W)
    labels = tgt_labels.astype(jnp.int32).reshape(bs, 1, T)

    grid = (bs, num_h_blocks)
    out = pl.pallas_call(
        functools.partial(_matcher_kernel, num_h_blocks, Q, T),
        grid=grid,
        in_specs=[
            pl.BlockSpec((1, Q, C), lambda b, h: (b, 0, 0)),
            pl.BlockSpec((1, 1, T), lambda b, h: (b, 0, 0)),
            pl.BlockSpec((Q, BH, W), lambda b, h: (b, h, 0)),
            pl.BlockSpec((T, BH, W), lambda b, h: (b, h, 0)),
        ],
        out_specs=pl.BlockSpec((1, Q, T), lambda b, h: (b, 0, 0)),
        out_shape=jax.ShapeDtypeStruct((bs, Q, T), jnp.float32),
        scratch_shapes=[
            pltpu.VMEM((3 * Q + 4, T + 1), jnp.float32),
        ],
        compiler_params=pltpu.CompilerParams(
            dimension_semantics=("parallel", "arbitrary")),
    )(pred_logits, labels, x3, y3)
    return out


# speedup vs baseline: 2.7844x; 1.0451x over previous
import functools

import jax
import jax.numpy as jnp
from jax.experimental import pallas as pl
from jax.experimental.pallas import tpu as pltpu

EPS = 1e-6
BH = 128


def _matcher_kernel(num_h_blocks, Q, T, logits_ref, labels_ref, x_ref, y_ref,
                    out_ref, acc):
    h = pl.program_id(1)

    @pl.when(h == 0)
    def _init():
        acc[...] = jnp.zeros_like(acc)

    xb = x_ref[...]
    yb = y_ref[...]
    W = xb.shape[-1]

    e = jnp.exp2(jnp.abs(xb) * (-1.4426950408889634))
    one_p_e = 1.0 + e
    sp = jnp.maximum(xb, 0.0) + jnp.log(one_p_e)
    sg = jnp.where(xb >= 0.0, 1.0, e) / one_p_e

    x16 = xb.astype(jnp.bfloat16)
    sg16 = sg.astype(jnp.bfloat16)
    sp16 = sp.astype(jnp.bfloat16)
    y16 = yb.astype(jnp.bfloat16)

    lhs = jnp.concatenate(
        [x16, sg16, sp16, jnp.ones((4, BH, W), jnp.bfloat16)], axis=0)
    rhs = jnp.concatenate(
        [y16, jnp.ones((1, BH, W), jnp.bfloat16)], axis=0)
    lhs_t = jax.lax.transpose(lhs, (1, 0, 2))
    rhs_t = jax.lax.transpose(rhs, (1, 0, 2))
    lhs_f = jnp.concatenate([lhs_t[hh] for hh in range(BH)], axis=1)
    rhs_f = jnp.concatenate([rhs_t[hh] for hh in range(BH)], axis=1)
    contract = (((1,), (1,)), ((), ()))
    acc[...] += jax.lax.dot_general(
        lhs_f, rhs_f, contract, preferred_element_type=jnp.float32)

    @pl.when(h == num_h_blocks - 1)
    def _finalize():
        inv_p = 1.0 / jnp.float32(num_h_blocks * BH * W)
        logits = logits_ref[0]
        m = jnp.max(logits, axis=-1, keepdims=True)
        ex = jnp.exp(logits - m)
        prob = ex / jnp.sum(ex, axis=-1, keepdims=True)
        lab = labels_ref[0]
        num_classes = logits.shape[-1]
        iota_c = jax.lax.broadcasted_iota(jnp.int32, (num_classes, T), 0)
        onehot = (iota_c == lab).astype(jnp.float32)
        cost_class = jax.lax.dot_general(
            prob, onehot, (((1,), (0,)), ((), ())),
            preferred_element_type=jnp.float32)

        xy = acc[0:Q, 0:T]
        sgy = acc[Q:2 * Q, 0:T]
        sg_sum = acc[Q:2 * Q, T:T + 1]
        sp_sum = acc[2 * Q:3 * Q, T:T + 1]
        y_sum = acc[3 * Q:3 * Q + 1, 0:T]

        bce = (sp_sum - xy) * inv_p
        denom = sg_sum + y_sum + EPS
        dice = 1.0 - 2.0 * sgy / denom
        out_ref[0] = bce + dice - cost_class


def kernel(pred_logits, pred_masks, tgt_labels, tgt_masks):
    bs, Q, C = pred_logits.shape
    T = tgt_masks.shape[1]
    H, W = pred_masks.shape[-2:]
    num_h_blocks = H // BH

    x3 = pred_masks.reshape(bs * Q, H, W)
    y3 = tgt_masks.reshape(bs * T, H, W)
    labels = tgt_labels.astype(jnp.int32).reshape(bs, 1, T)

    grid = (bs, num_h_blocks)
    out = pl.pallas_call(
        functools.partial(_matcher_kernel, num_h_blocks, Q, T),
        grid=grid,
        in_specs=[
            pl.BlockSpec((1, Q, C), lambda b, h: (b, 0, 0)),
            pl.BlockSpec((1, 1, T), lambda b, h: (b, 0, 0)),
            pl.BlockSpec((Q, BH, W), lambda b, h: (b, h, 0)),
            pl.BlockSpec((T, BH, W), lambda b, h: (b, h, 0)),
        ],
        out_specs=pl.BlockSpec((1, Q, T), lambda b, h: (b, 0, 0)),
        out_shape=jax.ShapeDtypeStruct((bs, Q, T), jnp.float32),
        scratch_shapes=[
            pltpu.VMEM((3 * Q + 4, T + 1), jnp.float32),
        ],
        compiler_params=pltpu.CompilerParams(
            dimension_semantics=("parallel", "arbitrary")),
    )(pred_logits, labels, x3, y3)
    return out


# speedup vs baseline: 2.8091x; 1.0089x over previous
import functools

import jax
import jax.numpy as jnp
from jax.experimental import pallas as pl
from jax.experimental.pallas import tpu as pltpu

EPS = 1e-6
BH = 64


def _matcher_kernel(num_h_blocks, Q, T, logits_ref, labels_ref, x_ref, y_ref,
                    out_ref, acc):
    h = pl.program_id(1)

    @pl.when(h == 0)
    def _init():
        acc[...] = jnp.zeros_like(acc)

    xb = x_ref[...]
    yb = y_ref[...]
    W = xb.shape[-1]

    e = jnp.exp2(jnp.abs(xb) * (-1.4426950408889634))
    one_p_e = 1.0 + e
    sp = jnp.maximum(xb, 0.0) + jnp.log(one_p_e)
    sg = jnp.where(xb >= 0.0, 1.0, e) / one_p_e

    x16 = xb.astype(jnp.bfloat16)
    sg16 = sg.astype(jnp.bfloat16)
    sp16 = sp.astype(jnp.bfloat16)
    y16 = yb.astype(jnp.bfloat16)

    lhs = jnp.concatenate(
        [x16, sg16, sp16, jnp.ones((4, BH, W), jnp.bfloat16)], axis=0)
    rhs = jnp.concatenate(
        [y16, jnp.ones((1, BH, W), jnp.bfloat16)], axis=0)
    lhs_t = jax.lax.transpose(lhs, (1, 0, 2))
    rhs_t = jax.lax.transpose(rhs, (1, 0, 2))
    lhs_f = jnp.concatenate([lhs_t[hh] for hh in range(BH)], axis=1)
    rhs_f = jnp.concatenate([rhs_t[hh] for hh in range(BH)], axis=1)
    contract = (((1,), (1,)), ((), ()))
    acc[...] += jax.lax.dot_general(
        lhs_f, rhs_f, contract, preferred_element_type=jnp.float32)

    @pl.when(h == num_h_blocks - 1)
    def _finalize():
        inv_p = 1.0 / jnp.float32(num_h_blocks * BH * W)
        logits = logits_ref[0]
        m = jnp.max(logits, axis=-1, keepdims=True)
        ex = jnp.exp(logits - m)
        prob = ex / jnp.sum(ex, axis=-1, keepdims=True)
        lab = labels_ref[0]
        num_classes = logits.shape[-1]
        iota_c = jax.lax.broadcasted_iota(jnp.int32, (num_classes, T), 0)
        onehot = (iota_c == lab).astype(jnp.float32)
        cost_class = jax.lax.dot_general(
            prob, onehot, (((1,), (0,)), ((), ())),
            preferred_element_type=jnp.float32)

        xy = acc[0:Q, 0:T]
        sgy = acc[Q:2 * Q, 0:T]
        sg_sum = acc[Q:2 * Q, T:T + 1]
        sp_sum = acc[2 * Q:3 * Q, T:T + 1]
        y_sum = acc[3 * Q:3 * Q + 1, 0:T]

        bce = (sp_sum - xy) * inv_p
        denom = sg_sum + y_sum + EPS
        dice = 1.0 - 2.0 * sgy / denom
        out_ref[0] = bce + dice - cost_class


def kernel(pred_logits, pred_masks, tgt_labels, tgt_masks):
    bs, Q, C = pred_logits.shape
    T = tgt_masks.shape[1]
    H, W = pred_masks.shape[-2:]
    num_h_blocks = H // BH

    x3 = pred_masks.reshape(bs * Q, H, W)
    y3 = tgt_masks.reshape(bs * T, H, W)
    labels = tgt_labels.astype(jnp.int32).reshape(bs, 1, T)

    grid = (bs, num_h_blocks)
    out = pl.pallas_call(
        functools.partial(_matcher_kernel, num_h_blocks, Q, T),
        grid=grid,
        in_specs=[
            pl.BlockSpec((1, Q, C), lambda b, h: (b, 0, 0)),
            pl.BlockSpec((1, 1, T), lambda b, h: (b, 0, 0)),
            pl.BlockSpec((Q, BH, W), lambda b, h: (b, h, 0)),
            pl.BlockSpec((T, BH, W), lambda b, h: (b, h, 0)),
        ],
        out_specs=pl.BlockSpec((1, Q, T), lambda b, h: (b, 0, 0)),
        out_shape=jax.ShapeDtypeStruct((bs, Q, T), jnp.float32),
        scratch_shapes=[
            pltpu.VMEM((3 * Q + 4, T + 1), jnp.float32),
        ],
        compiler_params=pltpu.CompilerParams(
            dimension_semantics=("parallel", "arbitrary")),
    )(pred_logits, labels, x3, y3)
    return out


# speedup vs baseline: 3.1888x; 1.1352x over previous
import functools

import jax
import jax.numpy as jnp
from jax.experimental import pallas as pl
from jax.experimental.pallas import tpu as pltpu

EPS = 1e-6
BH = 64
LOG2E = 1.4426950408889634


def _matcher_kernel(num_h_blocks, Q, T, logits_ref, labels_ref, x_ref, y_ref,
                    out_ref, acc_xy, acc_sg, acc_sp, acc_ys, xflat, yflat):
    h = pl.program_id(1)

    @pl.when(h == 0)
    def _init():
        acc_xy[...] = jnp.zeros_like(acc_xy)
        acc_sg[...] = jnp.zeros_like(acc_sg)
        acc_sp[...] = jnp.zeros_like(acc_sp)
        acc_ys[...] = jnp.zeros_like(acc_ys)

    W = x_ref.shape[-1]
    p_blk = BH * W
    xflat[...] = x_ref[...].reshape(Q, p_blk)
    yflat[...] = y_ref[...].reshape(T, p_blk)
    xb = xflat[...]
    yb = yflat[...]

    e = jnp.exp2(jnp.abs(xb) * (-LOG2E))
    one_p_e = 1.0 + e
    sp = jnp.maximum(xb, 0.0) + jnp.log(one_p_e)
    sg = jnp.where(xb >= 0.0, 1.0, e) / one_p_e

    x16 = xb.astype(jnp.bfloat16)
    sg16 = sg.astype(jnp.bfloat16)
    sp16 = sp.astype(jnp.bfloat16)
    rhs = jnp.concatenate(
        [yb.astype(jnp.bfloat16), jnp.ones((1, p_blk), jnp.bfloat16)],
        axis=0)
    ones8 = jnp.ones((8, p_blk), jnp.bfloat16)

    contract = (((1,), (1,)), ((), ()))
    acc_xy[...] += jax.lax.dot_general(
        x16, rhs, contract, preferred_element_type=jnp.float32)
    acc_sg[...] += jax.lax.dot_general(
        sg16, rhs, contract, preferred_element_type=jnp.float32)
    acc_sp[...] += jax.lax.dot_general(
        ones8, sp16, contract, preferred_element_type=jnp.float32)
    acc_ys[...] += jax.lax.dot_general(
        ones8, rhs, contract, preferred_element_type=jnp.float32)

    @pl.when(h == num_h_blocks - 1)
    def _finalize():
        inv_p = 1.0 / jnp.float32(num_h_blocks * BH * W)
        logits = logits_ref[0]
        m = jnp.max(logits, axis=-1, keepdims=True)
        ex = jnp.exp(logits - m)
        prob = ex / jnp.sum(ex, axis=-1, keepdims=True)
        lab = labels_ref[0]
        num_classes = logits.shape[-1]
        iota_c = jax.lax.broadcasted_iota(jnp.int32, (num_classes, T), 0)
        onehot = (iota_c == lab).astype(jnp.float32)
        cost_class = jax.lax.dot_general(
            prob, onehot, (((1,), (0,)), ((), ())),
            preferred_element_type=jnp.float32)

        xy = acc_xy[:, 0:T]
        sgy = acc_sg[:, 0:T]
        sg_sum = acc_sg[:, T:T + 1]
        sp_sum = jax.lax.transpose(acc_sp[0:1, :], (1, 0))
        y_sum = acc_ys[0:1, 0:T]

        bce = (sp_sum - xy) * inv_p
        denom = sg_sum + y_sum + EPS
        dice = 1.0 - 2.0 * sgy / denom
        out_ref[0] = bce + dice - cost_class


def kernel(pred_logits, pred_masks, tgt_labels, tgt_masks):
    bs, Q, C = pred_logits.shape
    T = tgt_masks.shape[1]
    H, W = pred_masks.shape[-2:]
    num_h_blocks = H // BH

    x3 = pred_masks.reshape(bs * Q, H, W)
    y3 = tgt_masks.reshape(bs * T, H, W)
    labels = tgt_labels.astype(jnp.int32).reshape(bs, 1, T)

    grid = (bs, num_h_blocks)
    out = pl.pallas_call(
        functools.partial(_matcher_kernel, num_h_blocks, Q, T),
        grid=grid,
        in_specs=[
            pl.BlockSpec((1, Q, C), lambda b, h: (b, 0, 0)),
            pl.BlockSpec((1, 1, T), lambda b, h: (b, 0, 0)),
            pl.BlockSpec((Q, BH, W), lambda b, h: (b, h, 0)),
            pl.BlockSpec((T, BH, W), lambda b, h: (b, h, 0)),
        ],
        out_specs=pl.BlockSpec((1, Q, T), lambda b, h: (b, 0, 0)),
        out_shape=jax.ShapeDtypeStruct((bs, Q, T), jnp.float32),
        scratch_shapes=[
            pltpu.VMEM((Q, T + 1), jnp.float32),
            pltpu.VMEM((Q, T + 1), jnp.float32),
            pltpu.VMEM((8, Q), jnp.float32),
            pltpu.VMEM((8, T + 1), jnp.float32),
            pltpu.VMEM((Q, BH * W), jnp.float32),
            pltpu.VMEM((T, BH * W), jnp.float32),
        ],
        compiler_params=pltpu.CompilerParams(
            dimension_semantics=("parallel", "arbitrary"),
            flags={"XLA_TPU_STORE_TO_LOAD_FORWARDING_WINDOW": 16384}),
    )(pred_logits, labels, x3, y3)
    return out


# speedup vs baseline: 3.3057x; 1.0367x over previous
import functools

import jax
import jax.numpy as jnp
from jax.experimental import pallas as pl
from jax.experimental.pallas import tpu as pltpu

EPS = 1e-6
BH = 64
LOG2E = 1.4426950408889634


def _matcher_kernel(num_h_blocks, Q, T, logits_ref, labels_ref, x_ref, y_ref,
                    out_ref, acc_xy, acc_sg, acc_sp, acc_ys, xflat):
    h = pl.program_id(1)

    @pl.when(h == 0)
    def _init():
        acc_xy[...] = jnp.zeros_like(acc_xy)
        acc_sg[...] = jnp.zeros_like(acc_sg)
        acc_sp[...] = jnp.zeros_like(acc_sp)
        acc_ys[...] = jnp.zeros_like(acc_ys)

    W = x_ref.shape[-1]
    p_blk = BH * W
    xflat[...] = x_ref[...].reshape(Q, p_blk)
    xb = xflat[...]

    e = jnp.exp2(jnp.abs(xb) * (-LOG2E))
    one_p_e = 1.0 + e
    sp = jnp.maximum(xb, 0.0) + jnp.log(one_p_e)
    sg = jnp.where(xb >= 0.0, 1.0, e) / one_p_e

    x16 = xb.astype(jnp.bfloat16)
    sg16 = sg.astype(jnp.bfloat16)
    sp16 = sp.astype(jnp.bfloat16)
    y16_3d = jnp.concatenate(
        [y_ref[...].astype(jnp.bfloat16), jnp.ones((1, BH, W), jnp.bfloat16)],
        axis=0)
    y_t = jax.lax.transpose(y16_3d, (1, 0, 2))
    rhs = jnp.concatenate(
        [y_t[hh] for hh in range(BH)], axis=1)
    ones8 = jnp.ones((8, p_blk), jnp.bfloat16)

    contract = (((1,), (1,)), ((), ()))
    acc_xy[...] += jax.lax.dot_general(
        x16, rhs, contract, preferred_element_type=jnp.float32)
    acc_sg[...] += jax.lax.dot_general(
        sg16, rhs, contract, preferred_element_type=jnp.float32)
    acc_sp[...] += jax.lax.dot_general(
        ones8, sp16, contract, preferred_element_type=jnp.float32)
    acc_ys[...] += jax.lax.dot_general(
        ones8, rhs, contract, preferred_element_type=jnp.float32)

    @pl.when(h == num_h_blocks - 1)
    def _finalize():
        inv_p = 1.0 / jnp.float32(num_h_blocks * BH * W)
        logits = logits_ref[0]
        m = jnp.max(logits, axis=-1, keepdims=True)
        ex = jnp.exp(logits - m)
        prob = ex / jnp.sum(ex, axis=-1, keepdims=True)
        lab = labels_ref[0]
        num_classes = logits.shape[-1]
        iota_c = jax.lax.broadcasted_iota(jnp.int32, (num_classes, T), 0)
        onehot = (iota_c == lab).astype(jnp.float32)
        cost_class = jax.lax.dot_general(
            prob, onehot, (((1,), (0,)), ((), ())),
            preferred_element_type=jnp.float32)

        xy = acc_xy[:, 0:T]
        sgy = acc_sg[:, 0:T]
        sg_sum = acc_sg[:, T:T + 1]
        sp_sum = jax.lax.transpose(acc_sp[0:1, :], (1, 0))
        y_sum = acc_ys[0:1, 0:T]

        bce = (sp_sum - xy) * inv_p
        denom = sg_sum + y_sum + EPS
        dice = 1.0 - 2.0 * sgy / denom
        out_ref[0] = bce + dice - cost_class


def kernel(pred_logits, pred_masks, tgt_labels, tgt_masks):
    bs, Q, C = pred_logits.shape
    T = tgt_masks.shape[1]
    H, W = pred_masks.shape[-2:]
    num_h_blocks = H // BH

    x3 = pred_masks.reshape(bs * Q, H, W)
    y3 = tgt_masks.reshape(bs * T, H, W)
    labels = tgt_labels.astype(jnp.int32).reshape(bs, 1, T)

    grid = (bs, num_h_blocks)
    out = pl.pallas_call(
        functools.partial(_matcher_kernel, num_h_blocks, Q, T),
        grid=grid,
        in_specs=[
            pl.BlockSpec((1, Q, C), lambda b, h: (b, 0, 0)),
            pl.BlockSpec((1, 1, T), lambda b, h: (b, 0, 0)),
            pl.BlockSpec((Q, BH, W), lambda b, h: (b, h, 0)),
            pl.BlockSpec((T, BH, W), lambda b, h: (b, h, 0)),
        ],
        out_specs=pl.BlockSpec((1, Q, T), lambda b, h: (b, 0, 0)),
        out_shape=jax.ShapeDtypeStruct((bs, Q, T), jnp.float32),
        scratch_shapes=[
            pltpu.VMEM((Q, T + 1), jnp.float32),
            pltpu.VMEM((Q, T + 1), jnp.float32),
            pltpu.VMEM((8, Q), jnp.float32),
            pltpu.VMEM((8, T + 1), jnp.float32),
            pltpu.VMEM((Q, BH * W), jnp.float32),
        ],
        compiler_params=pltpu.CompilerParams(
            dimension_semantics=("parallel", "arbitrary"),
            flags={"XLA_TPU_STORE_TO_LOAD_FORWARDING_WINDOW": 16384}),
    )(pred_logits, labels, x3, y3)
    return out
